# baseline (speedup 1.0000x reference)
import functools

import jax
import jax.numpy as jnp
from jax import lax
from jax.experimental import pallas as pl
from jax.experimental.pallas import tpu as pltpu
from jax.experimental.pallas import tpu_sc as plsc

BUCKETS = 1000000
DIM = 64
PDIM = 128
LANES = 16
NUM_CORES = 2
NUM_SUBCORES = 16
NUM_WORKERS = NUM_CORES * NUM_SUBCORES
HALO = 8
W = 128


def _sc_embed(token_flat, scale_vec, table_pad, *, b, t):
    bt = b * t
    chunk = bt // NUM_WORKERS
    n_grp = chunk // LANES
    n_win = chunk // W
    chunks_per_row = t // chunk
    mod = BUCKETS - 1

    mesh = plsc.VectorSubcoreMesh(core_axis_name="c", subcore_axis_name="s")

    @functools.partial(
        pl.kernel,
        out_type=jax.ShapeDtypeStruct((bt, DIM), jnp.float32),
        mesh=mesh,
        scratch_types=[
            pltpu.VMEM((HALO + chunk,), jnp.int32),
            pltpu.VMEM((n_win, W), jnp.int32),
            pltpu.VMEM((2, W, PDIM), jnp.float32),
            pltpu.VMEM((2, W, DIM), jnp.float32),
            pltpu.VMEM((LANES,), jnp.float32),
            pltpu.SemaphoreType.DMA,
            pltpu.SemaphoreType.DMA,
        ],
    )
    def body(tok_hbm, scale_hbm, table_hbm, out_hbm,
             tok_v, idx_v, rows_v, orows_v, scale_v, gsem, osem):
        wid = lax.axis_index("s") * NUM_CORES + lax.axis_index("c")
        cpos = (wid % chunks_per_row) * chunk
        base = wid * chunk
        at_row_start = cpos == 0

        pltpu.sync_copy(scale_hbm, scale_v)

        @pl.when(at_row_start)
        def _():
            pltpu.sync_copy(tok_hbm.at[pl.ds(base, chunk)],
                            tok_v.at[pl.ds(HALO, chunk)])

        @pl.when(jnp.logical_not(at_row_start))
        def _():
            pltpu.sync_copy(tok_hbm.at[pl.ds(base - HALO, HALO + chunk)],
                            tok_v)

        lanes = lax.iota(jnp.int32, LANES)
        pos_in_row = cpos + lanes

        def hash_group(i, _):
            q = i * LANES
            t2 = tok_v[pl.ds(q + HALO, LANES)]
            t1 = tok_v[pl.ds(q + HALO - 1, LANES)]
            t0 = tok_v[pl.ds(q + HALO - 2, LANES)]
            h = 131071 * t2 + 524287 * t1 + 8191 * t0
            m = h % mod
            m = jnp.where(pos_in_row + q < 2, mod, m)
            idx_v[i // (W // LANES), pl.ds((i % (W // LANES)) * LANES,
                                           LANES)] = m
            return 0

        lax.fori_loop(0, n_grp, hash_group, 0, unroll=8)

        sv = scale_v[...]

        def fire(j, buf):
            pltpu.async_copy(table_hbm.at[idx_v.at[j]], rows_v.at[buf], gsem)

        def wait_gather(buf):
            pltpu.make_async_copy(table_hbm.at[idx_v.at[0]],
                                  rows_v.at[buf], gsem).wait()

        def scale_rows(buf):
            def one(r, _):
                for c in range(DIM // LANES):
                    orows_v[buf, r, pl.ds(c * LANES, LANES)] = (
                        rows_v[buf, r, pl.ds(c * LANES, LANES)] * sv)
                return 0
            lax.fori_loop(0, W, one, 0, unroll=4)

        def put(j, buf):
            return pltpu.async_copy(
                orows_v.at[buf],
                out_hbm.at[pl.ds(base + j * W, W)], osem)

        fire(0, 0)

        def step(jj, _):
            j0 = jj * 2
            fire(j0 + 1, 1)
            wait_gather(0)
            scale_rows(0)
            put(j0, 0).wait()

            @pl.when(j0 + 2 < n_win)
            def _():
                fire(j0 + 2, 0)

            wait_gather(1)
            scale_rows(1)
            put(j0 + 1, 1).wait()
            return 0

        lax.fori_loop(0, n_win // 2, step, 0)

    return body(token_flat, scale_vec, table_pad)


def kernel(token_ids, embed_weight, scale):
    b, t = token_ids.shape
    table_pad = jnp.pad(embed_weight, ((0, 0), (0, PDIM - DIM)))
    scale_vec = jnp.full((LANES,), scale, dtype=jnp.float32)
    tok_flat = token_ids.reshape(b * t).astype(jnp.int32)
    out = _sc_embed(tok_flat, scale_vec, table_pad, b=b, t=t)
    return out.reshape(b, t, DIM)
